# Initial kernel scaffold; baseline (speedup 1.0000x reference)
import jax
import jax.numpy as jnp
from jax.experimental import pallas as pl


def kernel(hs, batches, W, b):
    raise NotImplementedError("write your pallas kernel here")


# speedup vs baseline: 4.8180x; 4.8180x over previous
import functools

import jax
import jax.numpy as jnp
from jax import lax
from jax.experimental import pallas as pl
from jax.experimental.pallas import tpu as pltpu
from jax.experimental.pallas import tpu_sc as plsc

L = 4
N = 100000
D = 128
B = 512

ROWS = L * N
ROWS_PER_CORE = 2 * N
SPAN = 12504
SPAN_LAST = ROWS_PER_CORE - 15 * SPAN
CHUNK = 128
NCHUNKS = 98
ACC_ROWS = 2 * B + 1


def _sc_body(hs_ref, bat_ref, out_ref, idx_buf, data_buf, stage, acc):
    c = lax.axis_index("c")
    s = lax.axis_index("s")

    base = c * ROWS_PER_CORE + s * SPAN
    span = jnp.where(s == 15, SPAN_LAST, SPAN)
    tail_base_off = span - CHUNK
    tail_mask_below = jnp.where(s == 15, CHUNK - (SPAN_LAST - 97 * CHUNK),
                                CHUNK - (SPAN - 97 * CHUNK))
    boundary = c * ROWS_PER_CORE + N

    zeros16 = jnp.zeros((16,), jnp.float32)
    for r in range(16):
        for g in range(8):
            stage[r, pl.ds(g * 16, 16)] = zeros16
    for k in range(4):
        pltpu.sync_copy(stage.at[pl.ds(0, 16), :],
                        acc.at[pl.ds(s * 64 + k * 16, 16), :])

    @pl.when(s == 0)
    def _zero_dump():
        pltpu.sync_copy(stage.at[pl.ds(0, 1), :], acc.at[pl.ds(2 * B, 1), :])

    plsc.subcore_barrier()

    def chunk_step(j, carry):
        off = jnp.where(j == NCHUNKS - 1, tail_base_off, j * CHUNK)
        cb = base + off
        mask_below = jnp.where(j == NCHUNKS - 1, tail_mask_below, 0)
        pltpu.sync_copy(bat_ref.at[pl.ds(cb, CHUNK)], idx_buf)
        pltpu.sync_copy(hs_ref.at[pl.ds(cb, CHUNK), :], data_buf)
        for g in range(8):
            lane = g * 16 + lax.iota(jnp.int32, 16)
            v = idx_buf[pl.ds(g * 16, 16)]
            v = jnp.where(cb + lane >= boundary, v + B, v)
            v = jnp.where(lane < mask_below, 2 * B, v)
            idx_buf[pl.ds(g * 16, 16)] = v
        pltpu.sync_copy(data_buf, acc.at[idx_buf], add=True)
        return carry

    lax.fori_loop(0, NCHUNKS, chunk_step, 0)

    plsc.subcore_barrier()

    out_base = c * 2 * B + s * 64
    for k in range(2):
        pltpu.sync_copy(acc.at[pl.ds(s * 64 + k * 32, 32), :], stage)
        pltpu.sync_copy(stage, out_ref.at[pl.ds(out_base + k * 32, 32), :])


_sc_segsum = pl.kernel(
    _sc_body,
    out_type=jax.ShapeDtypeStruct((2 * 2 * B, D), jnp.float32),
    mesh=plsc.VectorSubcoreMesh(core_axis_name="c", subcore_axis_name="s"),
    scratch_types=[
        pltpu.VMEM((CHUNK,), jnp.int32),
        pltpu.VMEM((CHUNK, D), jnp.float32),
        pltpu.VMEM((32, D), jnp.float32),
        pltpu.VMEM_SHARED((ACC_ROWS, D), jnp.float32),
    ],
)


def _proj_body(x_ref, w_ref, b_ref, o_ref):
    w = w_ref[...]
    r = jnp.broadcast_to(b_ref[...], (B, D))
    for l in range(L):
        x = x_ref[pl.ds(l * B, B), :]
        wl = w[:, l * D:(l + 1) * D]
        r = r + lax.dot_general(x, wl, (((1,), (1,)), ((), ())),
                                preferred_element_type=jnp.float32)
    o_ref[...] = r


def _project(parts, W, b2):
    return pl.pallas_call(
        _proj_body,
        out_shape=jax.ShapeDtypeStruct((B, D), jnp.float32),
    )(parts, W, b2)


@jax.jit
def kernel(hs, batches, W, b):
    hs2 = hs.reshape(ROWS, D)
    bat2 = batches.reshape(ROWS).astype(jnp.int32)
    parts = _sc_segsum(hs2, bat2)
    return _project(parts, W, b.reshape(1, D))


# speedup vs baseline: 7.8041x; 1.6198x over previous
import jax
import jax.numpy as jnp
from jax import lax
from jax.experimental import pallas as pl
from jax.experimental.pallas import tpu as pltpu
from jax.experimental.pallas import tpu_sc as plsc

L = 4
N = 100000
D = 128
B = 512

ROWS = L * N
CHUNK = 128
NCH = ROWS // CHUNK
NW = 32
BASE_CNT = NCH // NW
EXTRA = NCH - BASE_CNT * NW
MAXCNT = BASE_CNT + 1
ACC_ROWS = L * B


def _sc_body(hs_ref, bat_ref, out_ref, idx_dbuf, dbuf, stage, acc,
             sem_di0, sem_di1, sem_ld0, sem_ld1, sem_sc0, sem_sc1):
    c = lax.axis_index("c")
    s = lax.axis_index("s")
    wid = c * 16 + s
    cnt = jnp.where(wid < EXTRA, MAXCNT, BASE_CNT)
    start = BASE_CNT * wid + jnp.minimum(wid, EXTRA)

    sem_di = (sem_di0, sem_di1)
    sem_ld = (sem_ld0, sem_ld1)
    sem_sc = (sem_sc0, sem_sc1)
    dslot = (dbuf.at[0], dbuf.at[1])
    islot = (idx_dbuf.at[0], idx_dbuf.at[1])

    def loads_start(j, slot):
        base = (start + j) * CHUNK
        pltpu.async_copy(bat_ref.at[pl.ds(base, CHUNK)], islot[slot],
                         sem_di[slot])
        pltpu.async_copy(hs_ref.at[pl.ds(base, CHUNK), :], dslot[slot],
                         sem_ld[slot])

    def loads_wait(slot):
        pltpu.make_async_copy(bat_ref.at[pl.ds(0, CHUNK)], islot[slot],
                              sem_di[slot]).wait()
        pltpu.make_async_copy(hs_ref.at[pl.ds(0, CHUNK), :], dslot[slot],
                              sem_ld[slot]).wait()

    def fix_idx(j, slot):
        for g in range(8):
            lane = g * 16 + lax.iota(jnp.int32, 16)
            glob = (start + j) * CHUNK + lane
            v = idx_dbuf[slot, pl.ds(g * 16, 16)]
            v = v + jnp.where(glob >= N, B, 0)
            v = v + jnp.where(glob >= 2 * N, B, 0)
            v = v + jnp.where(glob >= 3 * N, B, 0)
            idx_dbuf[slot, pl.ds(g * 16, 16)] = v

    def scat_start(slot):
        pltpu.async_copy(dslot[slot], acc.at[islot[slot]],
                         sem_sc[slot], add=True)

    def scat_wait(slot):
        pltpu.make_async_copy(dslot[slot], acc.at[islot[slot]],
                              sem_sc[slot]).wait()

    loads_start(0, 0)

    zeros16 = jnp.zeros((16,), jnp.float32)
    for r in range(16):
        for g in range(8):
            stage[r, pl.ds(g * 16, 16)] = zeros16
    for k in range(8):
        pltpu.sync_copy(stage.at[pl.ds(0, 16), :],
                        acc.at[pl.ds(s * 128 + k * 16, 16), :])

    plsc.subcore_barrier()

    def pair_step(t, carry):
        j0 = 2 * t
        j1 = 2 * t + 1
        loads_wait(0)
        fix_idx(j0, 0)
        scat_start(0)

        @pl.when(t > 0)
        def _w1():
            scat_wait(1)

        @pl.when(j1 < cnt)
        def _h1():
            loads_start(j1, 1)
            loads_wait(1)
            fix_idx(j1, 1)
            scat_start(1)
            scat_wait(0)

            @pl.when(j1 + 1 < cnt)
            def _l2():
                loads_start(j1 + 1, 0)

        return carry

    lax.fori_loop(0, MAXCNT // 2, pair_step, 0)

    @pl.when(cnt == MAXCNT)
    def _drain1():
        scat_wait(1)

    @pl.when(cnt == BASE_CNT)
    def _drain0():
        scat_wait(0)

    plsc.subcore_barrier()

    for k in range(4):
        pltpu.sync_copy(acc.at[pl.ds(s * 128 + k * 32, 32), :], stage)
        pltpu.sync_copy(stage,
                        out_ref.at[pl.ds(c * ACC_ROWS + s * 128 + k * 32, 32), :])


_sc_segsum = pl.kernel(
    _sc_body,
    out_type=jax.ShapeDtypeStruct((2 * ACC_ROWS, D), jnp.float32),
    mesh=plsc.VectorSubcoreMesh(core_axis_name="c", subcore_axis_name="s"),
    scratch_types=[
        pltpu.VMEM((2, CHUNK), jnp.int32),
        pltpu.VMEM((2, CHUNK, D), jnp.float32),
        pltpu.VMEM((32, D), jnp.float32),
        pltpu.VMEM_SHARED((ACC_ROWS, D), jnp.float32),
        pltpu.SemaphoreType.DMA,
        pltpu.SemaphoreType.DMA,
        pltpu.SemaphoreType.DMA,
        pltpu.SemaphoreType.DMA,
        pltpu.SemaphoreType.DMA,
        pltpu.SemaphoreType.DMA,
    ],
)


def _proj_body(x_ref, w_ref, b_ref, o_ref):
    w = w_ref[...]
    r = jnp.broadcast_to(b_ref[...], (B, D))
    for l in range(L):
        x = x_ref[pl.ds(l * B, B), :] + x_ref[pl.ds(ACC_ROWS + l * B, B), :]
        wl = w[:, l * D:(l + 1) * D]
        r = r + lax.dot_general(x, wl, (((1,), (1,)), ((), ())),
                                preferred_element_type=jnp.float32)
    o_ref[...] = r


def _project(parts, W, b2):
    return pl.pallas_call(
        _proj_body,
        out_shape=jax.ShapeDtypeStruct((B, D), jnp.float32),
    )(parts, W, b2)


@jax.jit
def kernel(hs, batches, W, b):
    hs2 = hs.reshape(ROWS, D)
    bat2 = batches.reshape(ROWS).astype(jnp.int32)
    parts = _sc_segsum(hs2, bat2)
    return _project(parts, W, b.reshape(1, D))
